# baseline (speedup 1.0000x reference)
import functools

import jax
import jax.numpy as jnp
from jax import lax
from jax.experimental import pallas as pl
from jax.experimental.pallas import tpu as pltpu
from jax.experimental.pallas import tpu_sc as plsc

_NC = 2
_NS = 16
_NW = _NC * _NS
_CH = 1280


def _stage1_body(t, x4_ref, we4_ref, wself4_ref, table_ref, xself_ref):
    xb = x4_ref[...]
    for tt in range(t):
        table_ref[tt] = jnp.dot(xb, we4_ref[tt], preferred_element_type=jnp.float32)
    xself_ref[...] = jnp.dot(xb, wself4_ref[...], preferred_element_type=jnp.float32)


def _stage3_body(p0_ref, p1_ref, xs_ref, b_ref, wc_ref, bc_ref, out_ref):
    h = p0_ref[...] + p1_ref[...] + xs_ref[...] + b_ref[...]
    g = jax.nn.gelu(h)
    out_ref[...] = jnp.dot(g, wc_ref[...], preferred_element_type=jnp.float32) + bc_ref[...]


def _make_sc_agg(np_, n, e, t, c_out):
    nch = e // _CH
    rpt = (n // _NS) // 8 * 8
    tail = n - _NS * rpt
    mesh = plsc.VectorSubcoreMesh(core_axis_name="c", subcore_axis_name="s")

    @functools.partial(
        pl.kernel,
        mesh=mesh,
        out_type=jax.ShapeDtypeStruct((_NC * np_, c_out), jnp.float32),
        scratch_types=[
            pltpu.VMEM((_CH,), jnp.int32),
            pltpu.VMEM((_CH,), jnp.int32),
            pltpu.VMEM((_CH,), jnp.int32),
            pltpu.VMEM((_CH,), jnp.int32),
            pltpu.VMEM((_CH,), jnp.int32),
            pltpu.VMEM((_CH,), jnp.int32),
            pltpu.VMEM((_CH,), jnp.int32),
            pltpu.VMEM((_CH,), jnp.int32),
            pltpu.VMEM((_CH, c_out), jnp.float32),
            pltpu.VMEM((_CH, c_out), jnp.float32),
            pltpu.VMEM_SHARED((n, c_out), jnp.float32),
            pltpu.SemaphoreType.DMA,
            pltpu.SemaphoreType.DMA,
            pltpu.SemaphoreType.DMA,
            pltpu.SemaphoreType.DMA,
        ],
        compiler_params=pltpu.CompilerParams(use_tc_tiling_on_sc=False),
    )
    def sc_agg(table_hbm, src_hbm, et_hbm, dst_hbm, zeros_hbm, out_hbm,
               s0, s1, e0, e1, d0, d1, g0, g1, r0, r1, acc,
               si0, si1, sg0, sg1):
        sb, eb, db, gb = (s0, s1), (e0, e1), (d0, d1), (g0, g1)
        rb, si, sg = (r0, r1), (si0, si1), (sg0, sg1)
        cid = lax.axis_index("c")
        sid = lax.axis_index("s")
        wid = sid * _NC + cid

        pltpu.sync_copy(zeros_hbm.at[pl.ds(sid * rpt, rpt)],
                        acc.at[pl.ds(sid * rpt, rpt)])
        if tail:
            @pl.when(sid == _NS - 1)
            def _init_tail():
                pltpu.sync_copy(zeros_hbm.at[pl.ds(_NS * rpt, tail)],
                                acc.at[pl.ds(_NS * rpt, tail)])
        plsc.subcore_barrier()

        nk = (nch - wid + _NW - 1) // _NW

        def fire_idx(b, k):
            base = (wid + k * _NW) * _CH
            pltpu.async_copy(src_hbm.at[pl.ds(base, _CH)], sb[b], si[b])
            pltpu.async_copy(et_hbm.at[pl.ds(base, _CH)], eb[b], si[b])
            pltpu.async_copy(dst_hbm.at[pl.ds(base, _CH)], db[b], si[b])

        def wait_idx_fire_gather(b):
            pltpu.make_async_copy(src_hbm.at[pl.ds(0, _CH)], sb[b], si[b]).wait()
            pltpu.make_async_copy(et_hbm.at[pl.ds(0, _CH)], eb[b], si[b]).wait()
            pltpu.make_async_copy(dst_hbm.at[pl.ds(0, _CH)], db[b], si[b]).wait()
            for i in range(_CH // 16):
                sl = pl.ds(i * 16, 16)
                gb[b][sl] = eb[b][sl] * np_ + sb[b][sl]
            pltpu.async_copy(table_hbm.at[gb[b]], rb[b], sg[b])

        def wait_gather_scatter(b):
            pltpu.make_async_copy(table_hbm.at[gb[b]], rb[b], sg[b]).wait()
            pltpu.sync_copy(rb[b], acc.at[db[b]], add=True)

        fire_idx(0, 0)
        wait_idx_fire_gather(0)

        @pl.when(nk > 1)
        def _prefetch1():
            fire_idx(1, 1)

        def body(p, carry):
            k1 = 2 * p + 1
            k2 = 2 * p + 2
            k3 = 2 * p + 3

            @pl.when(k1 < nk)
            def _():
                wait_idx_fire_gather(1)

            wait_gather_scatter(0)

            @pl.when(k2 < nk)
            def _():
                fire_idx(0, k2)

            @pl.when(k1 < nk)
            def _():
                wait_gather_scatter(1)

            @pl.when(k2 < nk)
            def _():
                wait_idx_fire_gather(0)

            @pl.when(k3 < nk)
            def _():
                fire_idx(1, k3)

            return carry

        lax.fori_loop(0, (nk + 1) // 2, body, 0)
        plsc.subcore_barrier()
        pltpu.sync_copy(acc.at[pl.ds(sid * rpt, rpt)],
                        out_hbm.at[pl.ds(cid * np_ + sid * rpt, rpt)])
        if tail:
            @pl.when(sid == _NS - 1)
            def _drain_tail():
                pltpu.sync_copy(acc.at[pl.ds(_NS * rpt, tail)],
                                out_hbm.at[pl.ds(cid * np_ + _NS * rpt, tail)])

    return sc_agg


def kernel(x, edge_index, edge_type, W_edge, W_self, b, W_kl, b_kl, W_post, b_post):
    n, c_in = x.shape
    t, _, c_out = W_edge.shape
    e = edge_type.shape[0]
    embed = W_post.shape[0]
    avg_degree = 7.0
    pack = 128 // c_out

    blk = 2048
    np_ = 10240
    g = np_ // blk

    assert e % _CH == 0 and _CH % 16 == 0 and n % _NS == 0 and np_ % blk == 0

    we = W_edge / avg_degree
    we4 = jnp.stack([jax.scipy.linalg.block_diag(*([we[tt]] * pack))
                     for tt in range(t)])
    wself4 = jax.scipy.linalg.block_diag(*([W_self] * pack))
    wc = W_kl[:, :embed] @ W_post
    wc4 = jax.scipy.linalg.block_diag(*([wc] * pack))
    bc = b_kl[:embed] @ W_post + b_post
    bc4 = jnp.tile(bc, pack).reshape(1, 128)
    b4 = jnp.tile(b, pack).reshape(1, 128)

    x4 = jnp.pad(x, ((0, np_ - n), (0, 0))).reshape(np_ // 4, pack * c_in)

    table, xself = pl.pallas_call(
        functools.partial(_stage1_body, t),
        grid=(g,),
        in_specs=[
            pl.BlockSpec((blk // 4, pack * c_in), lambda i: (i, 0)),
            pl.BlockSpec((t, pack * c_in, 128), lambda i: (0, 0, 0)),
            pl.BlockSpec((pack * c_in, 128), lambda i: (0, 0)),
        ],
        out_specs=[
            pl.BlockSpec((t, blk // 4, 128), lambda i: (0, i, 0)),
            pl.BlockSpec((blk // 4, 128), lambda i: (i, 0)),
        ],
        out_shape=[
            jax.ShapeDtypeStruct((t, np_ // 4, 128), jnp.float32),
            jax.ShapeDtypeStruct((np_ // 4, 128), jnp.float32),
        ],
    )(x4, we4, wself4)
    table = table.reshape(t * np_, c_out)

    zeros = jnp.zeros((n, c_out), jnp.float32)
    partials = _make_sc_agg(np_, n, e, t, c_out)(
        table, edge_index[0], edge_type, edge_index[1], zeros)
    packed = partials.reshape(_NC * np_ // 4, 128)

    recon = pl.pallas_call(
        _stage3_body,
        grid=(g,),
        in_specs=[
            pl.BlockSpec((blk // 4, 128), lambda i: (i, 0)),
            pl.BlockSpec((blk // 4, 128), lambda i: (i, 0)),
            pl.BlockSpec((blk // 4, 128), lambda i: (i, 0)),
            pl.BlockSpec((1, 128), lambda i: (0, 0)),
            pl.BlockSpec((128, 128), lambda i: (0, 0)),
            pl.BlockSpec((1, 128), lambda i: (0, 0)),
        ],
        out_specs=pl.BlockSpec((blk // 4, 128), lambda i: (i, 0)),
        out_shape=jax.ShapeDtypeStruct((np_ // 4, 128), jnp.float32),
    )(packed[:np_ // 4], packed[np_ // 4:], xself, b4, wc4, bc4)
    return recon.reshape(np_, c_out)[:n]


# speedup vs baseline: 1.0085x; 1.0085x over previous
import functools

import jax
import jax.numpy as jnp
from jax import lax
from jax.experimental import pallas as pl
from jax.experimental.pallas import tpu as pltpu
from jax.experimental.pallas import tpu_sc as plsc

_NC = 2
_NS = 16
_NW = _NC * _NS
_CH = 640


def _stage1_body(t, x4_ref, we4_ref, wself4_ref, table_ref, xself_ref):
    xb = x4_ref[...]
    for tt in range(t):
        table_ref[tt] = jnp.dot(xb, we4_ref[tt], preferred_element_type=jnp.float32)
    xself_ref[...] = jnp.dot(xb, wself4_ref[...], preferred_element_type=jnp.float32)


def _stage3_body(p0_ref, p1_ref, xs_ref, b_ref, wc_ref, bc_ref, out_ref):
    h = p0_ref[...] + p1_ref[...] + xs_ref[...] + b_ref[...]
    g = jax.nn.gelu(h)
    out_ref[...] = jnp.dot(g, wc_ref[...], preferred_element_type=jnp.float32) + bc_ref[...]


def _make_sc_agg(np_, n, e, t, c_out):
    nch = e // _CH
    rpt = (n // _NS) // 8 * 8
    tail = n - _NS * rpt
    mesh = plsc.VectorSubcoreMesh(core_axis_name="c", subcore_axis_name="s")

    @functools.partial(
        pl.kernel,
        mesh=mesh,
        out_type=jax.ShapeDtypeStruct((_NC * np_, c_out), jnp.float32),
        scratch_types=[
            pltpu.VMEM((_CH,), jnp.int32),
            pltpu.VMEM((_CH,), jnp.int32),
            pltpu.VMEM((_CH,), jnp.int32),
            pltpu.VMEM((_CH,), jnp.int32),
            pltpu.VMEM((_CH,), jnp.int32),
            pltpu.VMEM((_CH,), jnp.int32),
            pltpu.VMEM((_CH,), jnp.int32),
            pltpu.VMEM((_CH,), jnp.int32),
            pltpu.VMEM((_CH, c_out), jnp.float32),
            pltpu.VMEM((_CH, c_out), jnp.float32),
            pltpu.VMEM_SHARED((n, c_out), jnp.float32),
            pltpu.SemaphoreType.DMA,
            pltpu.SemaphoreType.DMA,
            pltpu.SemaphoreType.DMA,
            pltpu.SemaphoreType.DMA,
        ],
        compiler_params=pltpu.CompilerParams(use_tc_tiling_on_sc=False),
    )
    def sc_agg(table_hbm, src_hbm, et_hbm, dst_hbm, zeros_hbm, out_hbm,
               s0, s1, e0, e1, d0, d1, g0, g1, r0, r1, acc,
               si0, si1, sg0, sg1):
        sb, eb, db, gb = (s0, s1), (e0, e1), (d0, d1), (g0, g1)
        rb, si, sg = (r0, r1), (si0, si1), (sg0, sg1)
        cid = lax.axis_index("c")
        sid = lax.axis_index("s")
        wid = sid * _NC + cid

        pltpu.sync_copy(zeros_hbm.at[pl.ds(sid * rpt, rpt)],
                        acc.at[pl.ds(sid * rpt, rpt)])
        if tail:
            @pl.when(sid == _NS - 1)
            def _init_tail():
                pltpu.sync_copy(zeros_hbm.at[pl.ds(_NS * rpt, tail)],
                                acc.at[pl.ds(_NS * rpt, tail)])
        plsc.subcore_barrier()

        nk = (nch - wid + _NW - 1) // _NW

        def fire_idx(b, k):
            base = (wid + k * _NW) * _CH
            pltpu.async_copy(src_hbm.at[pl.ds(base, _CH)], sb[b], si[b])
            pltpu.async_copy(et_hbm.at[pl.ds(base, _CH)], eb[b], si[b])
            pltpu.async_copy(dst_hbm.at[pl.ds(base, _CH)], db[b], si[b])

        def wait_idx_fire_gather(b):
            pltpu.make_async_copy(src_hbm.at[pl.ds(0, _CH)], sb[b], si[b]).wait()
            pltpu.make_async_copy(et_hbm.at[pl.ds(0, _CH)], eb[b], si[b]).wait()
            pltpu.make_async_copy(dst_hbm.at[pl.ds(0, _CH)], db[b], si[b]).wait()
            for i in range(_CH // 16):
                sl = pl.ds(i * 16, 16)
                gb[b][sl] = eb[b][sl] * np_ + sb[b][sl]
            pltpu.async_copy(table_hbm.at[gb[b]], rb[b], sg[b])

        def wait_gather_scatter(b):
            pltpu.make_async_copy(table_hbm.at[gb[b]], rb[b], sg[b]).wait()
            pltpu.sync_copy(rb[b], acc.at[db[b]], add=True)

        fire_idx(0, 0)
        wait_idx_fire_gather(0)

        @pl.when(nk > 1)
        def _prefetch1():
            fire_idx(1, 1)

        def body(p, carry):
            k1 = 2 * p + 1
            k2 = 2 * p + 2
            k3 = 2 * p + 3

            @pl.when(k1 < nk)
            def _():
                wait_idx_fire_gather(1)

            wait_gather_scatter(0)

            @pl.when(k2 < nk)
            def _():
                fire_idx(0, k2)

            @pl.when(k1 < nk)
            def _():
                wait_gather_scatter(1)

            @pl.when(k2 < nk)
            def _():
                wait_idx_fire_gather(0)

            @pl.when(k3 < nk)
            def _():
                fire_idx(1, k3)

            return carry

        lax.fori_loop(0, (nk + 1) // 2, body, 0)
        plsc.subcore_barrier()
        pltpu.sync_copy(acc.at[pl.ds(sid * rpt, rpt)],
                        out_hbm.at[pl.ds(cid * np_ + sid * rpt, rpt)])
        if tail:
            @pl.when(sid == _NS - 1)
            def _drain_tail():
                pltpu.sync_copy(acc.at[pl.ds(_NS * rpt, tail)],
                                out_hbm.at[pl.ds(cid * np_ + _NS * rpt, tail)])

    return sc_agg


def kernel(x, edge_index, edge_type, W_edge, W_self, b, W_kl, b_kl, W_post, b_post):
    n, c_in = x.shape
    t, _, c_out = W_edge.shape
    e = edge_type.shape[0]
    embed = W_post.shape[0]
    avg_degree = 7.0
    pack = 128 // c_out

    blk = 2048
    np_ = 10240
    g = np_ // blk

    assert e % _CH == 0 and _CH % 16 == 0 and n % _NS == 0 and np_ % blk == 0

    we = W_edge / avg_degree
    we4 = jnp.stack([jax.scipy.linalg.block_diag(*([we[tt]] * pack))
                     for tt in range(t)])
    wself4 = jax.scipy.linalg.block_diag(*([W_self] * pack))
    wc = W_kl[:, :embed] @ W_post
    wc4 = jax.scipy.linalg.block_diag(*([wc] * pack))
    bc = b_kl[:embed] @ W_post + b_post
    bc4 = jnp.tile(bc, pack).reshape(1, 128)
    b4 = jnp.tile(b, pack).reshape(1, 128)

    x4 = jnp.pad(x, ((0, np_ - n), (0, 0))).reshape(np_ // 4, pack * c_in)

    table, xself = pl.pallas_call(
        functools.partial(_stage1_body, t),
        grid=(g,),
        in_specs=[
            pl.BlockSpec((blk // 4, pack * c_in), lambda i: (i, 0)),
            pl.BlockSpec((t, pack * c_in, 128), lambda i: (0, 0, 0)),
            pl.BlockSpec((pack * c_in, 128), lambda i: (0, 0)),
        ],
        out_specs=[
            pl.BlockSpec((t, blk // 4, 128), lambda i: (0, i, 0)),
            pl.BlockSpec((blk // 4, 128), lambda i: (i, 0)),
        ],
        out_shape=[
            jax.ShapeDtypeStruct((t, np_ // 4, 128), jnp.float32),
            jax.ShapeDtypeStruct((np_ // 4, 128), jnp.float32),
        ],
    )(x4, we4, wself4)
    table = table.reshape(t * np_, c_out)

    zeros = jnp.zeros((n, c_out), jnp.float32)
    partials = _make_sc_agg(np_, n, e, t, c_out)(
        table, edge_index[0], edge_type, edge_index[1], zeros)
    packed = partials.reshape(_NC * np_ // 4, 128)

    recon = pl.pallas_call(
        _stage3_body,
        grid=(g,),
        in_specs=[
            pl.BlockSpec((blk // 4, 128), lambda i: (i, 0)),
            pl.BlockSpec((blk // 4, 128), lambda i: (i, 0)),
            pl.BlockSpec((blk // 4, 128), lambda i: (i, 0)),
            pl.BlockSpec((1, 128), lambda i: (0, 0)),
            pl.BlockSpec((128, 128), lambda i: (0, 0)),
            pl.BlockSpec((1, 128), lambda i: (0, 0)),
        ],
        out_specs=pl.BlockSpec((blk // 4, 128), lambda i: (i, 0)),
        out_shape=jax.ShapeDtypeStruct((np_ // 4, 128), jnp.float32),
    )(packed[:np_ // 4], packed[np_ // 4:], xself, b4, wc4, bc4)
    return recon.reshape(np_, c_out)[:n]
